# baseline (speedup 1.0000x reference)
import functools

import jax
import jax.numpy as jnp
from jax import lax
from jax.experimental import pallas as pl
from jax.experimental.pallas import tpu as pltpu
from jax.experimental.pallas import tpu_sc as plsc

TOKENS = 8192
D_MODEL = 2048
HIDDEN = 256
N_EXPERTS = 16
TILE = 2048

NC = 2
NS = 16
L = 16
NW = NC * NS
ROWS_PER_W = TOKENS // NW
BATCHES = ROWS_PER_W // L


def _logits_body(x_ref, w1_ref, b1_ref, w2_ref, b2_ref, out_ref):
    h = jnp.maximum(
        jnp.dot(x_ref[...], w1_ref[...], preferred_element_type=jnp.float32)
        + b1_ref[...],
        0.0,
    )
    out_ref[...] = (
        jnp.dot(h, w2_ref[...], preferred_element_type=jnp.float32) + b2_ref[...]
    )


def _tc_logits(x, W1, b1, W2, b2):
    grid = (TOKENS // TILE,)
    return pl.pallas_call(
        _logits_body,
        grid=grid,
        in_specs=[
            pl.BlockSpec((TILE, D_MODEL), lambda i: (i, 0)),
            pl.BlockSpec((D_MODEL, HIDDEN), lambda i: (0, 0)),
            pl.BlockSpec((1, HIDDEN), lambda i: (0, 0)),
            pl.BlockSpec((HIDDEN, N_EXPERTS), lambda i: (0, 0)),
            pl.BlockSpec((1, N_EXPERTS), lambda i: (0, 0)),
        ],
        out_specs=pl.BlockSpec((TILE, N_EXPERTS), lambda i: (i, 0)),
        out_shape=jax.ShapeDtypeStruct((TOKENS, N_EXPERTS), jnp.float32),
    )(x, W1, b1.reshape(1, HIDDEN), W2, b2.reshape(1, N_EXPERTS))


def _sc_route_body(logits_hbm, out_hbm, in_v, out_v):
    wid = lax.axis_index("s") * NC + lax.axis_index("c")
    base = wid * ROWS_PER_W
    pltpu.sync_copy(logits_hbm.at[pl.ds(base, ROWS_PER_W)], in_v)

    lane = lax.iota(jnp.int32, L)

    def batch(i, _):
        row_idx = i * L + lane
        cols = [
            plsc.load_gather(in_v, [row_idx, jnp.full((L,), e, jnp.int32)])
            for e in range(N_EXPERTS)
        ]
        m1 = cols[0]
        a1 = jnp.zeros((L,), jnp.int32)
        m2 = jnp.full((L,), -jnp.inf, jnp.float32)
        a2 = jnp.zeros((L,), jnp.int32)
        for e in range(1, N_EXPERTS):
            v = cols[e]
            ev = jnp.full((L,), e, jnp.int32)
            gt1 = v > m1
            gt2 = v > m2
            m2 = jnp.where(gt1, m1, jnp.where(gt2, v, m2))
            a2 = jnp.where(gt1, a1, jnp.where(gt2, ev, a2))
            m1 = jnp.where(gt1, v, m1)
            a1 = jnp.where(gt1, ev, a1)
        e2 = jnp.exp(m2 - m1)
        g1 = 1.0 / (1.0 + e2)
        g2 = e2 * g1
        zero = jnp.zeros((L,), jnp.float32)
        for e in range(N_EXPERTS):
            ev = jnp.full((L,), e, jnp.int32)
            col = jnp.where(a1 == ev, g1, zero) + jnp.where(a2 == ev, g2, zero)
            plsc.store_scatter(out_v, [row_idx, ev], col)
        return _

    lax.fori_loop(0, BATCHES, batch, None)
    pltpu.sync_copy(out_v, out_hbm.at[pl.ds(base, ROWS_PER_W)])


def _sc_route(logits):
    mesh = plsc.VectorSubcoreMesh(core_axis_name="c", subcore_axis_name="s")
    return pl.kernel(
        _sc_route_body,
        out_type=jax.ShapeDtypeStruct((TOKENS, N_EXPERTS), jnp.float32),
        mesh=mesh,
        scratch_types=[
            pltpu.VMEM((ROWS_PER_W, N_EXPERTS), jnp.float32),
            pltpu.VMEM((ROWS_PER_W, N_EXPERTS), jnp.float32),
        ],
        compiler_params=pltpu.CompilerParams(needs_layout_passes=False),
    )(logits)


@functools.partial(jax.jit, static_argnames=())
def kernel(x, W1, b1, W2, b2):
    logits = _tc_logits(x, W1, b1, W2, b2)
    return _sc_route(logits)


# speedup vs baseline: 1.5510x; 1.5510x over previous
import functools

import jax
import jax.numpy as jnp
from jax.experimental import pallas as pl

TOKENS = 8192
D_MODEL = 2048
HIDDEN = 256
N_EXPERTS = 16
F = 2
TILE = 1024
ROWS_F = TOKENS // F


def _gate_body(x_ref, w1_ref, b1_ref, w2_ref, b2_ref, out_ref):
    for f in range(F):
        h = jnp.maximum(
            jnp.dot(x_ref[f], w1_ref[...], preferred_element_type=jnp.float32)
            + b1_ref[...],
            0.0,
        )
        logits = (
            jnp.dot(h, w2_ref[...], preferred_element_type=jnp.float32)
            + b2_ref[...]
        )
        eidx = jax.lax.broadcasted_iota(jnp.int32, logits.shape, 1)
        m1 = jnp.max(logits, axis=-1, keepdims=True)
        a1 = jnp.min(
            jnp.where(logits == m1, eidx, N_EXPERTS), axis=-1, keepdims=True
        )
        masked = jnp.where(eidx == a1, -jnp.inf, logits)
        m2 = jnp.max(masked, axis=-1, keepdims=True)
        a2 = jnp.min(
            jnp.where(masked == m2, eidx, N_EXPERTS), axis=-1, keepdims=True
        )
        e2 = jnp.exp(m2 - m1)
        g1 = 1.0 / (1.0 + e2)
        g2 = e2 * g1
        out_ref[f] = jnp.where(eidx == a1, g1, jnp.where(eidx == a2, g2, 0.0))


@functools.partial(jax.jit, static_argnames=())
def kernel(x, W1, b1, W2, b2):
    grid = (ROWS_F // TILE,)
    out = pl.pallas_call(
        _gate_body,
        grid=grid,
        in_specs=[
            pl.BlockSpec((F, TILE, D_MODEL), lambda i: (0, i, 0)),
            pl.BlockSpec((D_MODEL, HIDDEN), lambda i: (0, 0)),
            pl.BlockSpec((1, HIDDEN), lambda i: (0, 0)),
            pl.BlockSpec((HIDDEN, N_EXPERTS), lambda i: (0, 0)),
            pl.BlockSpec((1, N_EXPERTS), lambda i: (0, 0)),
        ],
        out_specs=pl.BlockSpec((F, TILE, N_EXPERTS), lambda i: (0, i, 0)),
        out_shape=jax.ShapeDtypeStruct((F, ROWS_F, N_EXPERTS), jnp.float32),
    )(
        x.reshape(F, ROWS_F, D_MODEL),
        W1,
        b1.reshape(1, HIDDEN),
        W2,
        b2.reshape(1, N_EXPERTS),
    )
    return out.reshape(TOKENS, N_EXPERTS)


# speedup vs baseline: 1.6956x; 1.0932x over previous
import functools

import jax
import jax.numpy as jnp
from jax.experimental import pallas as pl

TOKENS = 8192
D_MODEL = 2048
HIDDEN = 256
N_EXPERTS = 16
TILE = 2048


def _gate_body(x_ref, w1_ref, b1_ref, w2_ref, b2_ref, out_ref):
    h = jnp.maximum(
        jnp.dot(x_ref[...], w1_ref[...], preferred_element_type=jnp.float32)
        + b1_ref[...],
        0.0,
    )
    logits = (
        jnp.dot(h, w2_ref[...], preferred_element_type=jnp.float32) + b2_ref[...]
    )
    eidx = jax.lax.broadcasted_iota(jnp.int32, logits.shape, 1)
    m1 = jnp.max(logits, axis=-1, keepdims=True)
    a1 = jnp.min(
        jnp.where(logits == m1, eidx, N_EXPERTS), axis=-1, keepdims=True
    )
    masked = jnp.where(eidx == a1, -jnp.inf, logits)
    m2 = jnp.max(masked, axis=-1, keepdims=True)
    a2 = jnp.min(
        jnp.where(masked == m2, eidx, N_EXPERTS), axis=-1, keepdims=True
    )
    e2 = jnp.exp(m2 - m1)
    g1 = 1.0 / (1.0 + e2)
    g2 = e2 * g1
    out_ref[...] = jnp.where(eidx == a1, g1, jnp.where(eidx == a2, g2, 0.0))


@functools.partial(jax.jit, static_argnames=())
def kernel(x, W1, b1, W2, b2):
    grid = (TOKENS // TILE,)
    return pl.pallas_call(
        _gate_body,
        grid=grid,
        in_specs=[
            pl.BlockSpec((TILE, D_MODEL), lambda i: (i, 0)),
            pl.BlockSpec((D_MODEL, HIDDEN), lambda i: (0, 0)),
            pl.BlockSpec((1, HIDDEN), lambda i: (0, 0)),
            pl.BlockSpec((HIDDEN, N_EXPERTS), lambda i: (0, 0)),
            pl.BlockSpec((1, N_EXPERTS), lambda i: (0, 0)),
        ],
        out_specs=pl.BlockSpec((TILE, N_EXPERTS), lambda i: (i, 0)),
        out_shape=jax.ShapeDtypeStruct((TOKENS, N_EXPERTS), jnp.float32),
    )(x, W1, b1.reshape(1, HIDDEN), W2, b2.reshape(1, N_EXPERTS))
